# baseline (speedup 1.0000x reference)
import functools

import numpy as np
import jax
import jax.numpy as jnp
from jax import lax
from jax.experimental import pallas as pl
from jax.experimental.pallas import tpu as pltpu
from jax.experimental.pallas import tpu_sc as plsc

_AVG = 16.0
_SILU_NORM = 0.5595081467
_SH_C = float(np.sqrt(3.0 / (4.0 * np.pi)))

_N = 10000
_E = 160000
_D = 128

_NW = 32
_G_GRP = 3 * 128
_G_SPAN = 39 * 128
_G_NG = 13


def _k1_gather(node_feats, senders):
    mesh = plsc.VectorSubcoreMesh(core_axis_name="c", subcore_axis_name="s")

    @functools.partial(
        pl.kernel,
        out_type=jax.ShapeDtypeStruct((_E, _D), jnp.float32),
        mesh=mesh,
        scratch_types=[
            pltpu.VMEM((_G_GRP,), jnp.int32),
            pltpu.VMEM((_G_GRP,), jnp.int32),
            pltpu.VMEM((_G_GRP, _D), jnp.float32),
            pltpu.VMEM((_G_GRP, _D), jnp.float32),
            pltpu.SemaphoreType.DMA,
            pltpu.SemaphoreType.DMA,
            pltpu.SemaphoreType.DMA,
            pltpu.SemaphoreType.DMA,
        ],
    )
    def k(nf_hbm, idx_hbm, out_hbm, idx0, idx1, rows0, rows1,
          semi0, semi1, semg0, semg1):
        idx_b = (idx0, idx1)
        rows_b = (rows0, rows1)
        semi_b = (semi0, semi1)
        semg_b = (semg0, semg1)
        wid = lax.axis_index("s") * 2 + lax.axis_index("c")
        base_w = wid * _G_SPAN

        def fire_idx(g, s):
            pltpu.make_async_copy(
                idx_hbm.at[pl.ds(base_w + g * _G_GRP, _G_GRP)],
                idx_b[s], semi_b[s]).start()

        def do_group(g, s):
            pltpu.make_async_copy(
                idx_hbm.at[pl.ds(base_w + g * _G_GRP, _G_GRP)],
                idx_b[s], semi_b[s]).wait()
            for j in range(3):
                pltpu.make_async_copy(
                    nf_hbm.at[idx_b[s].at[pl.ds(j * 128, 128)]],
                    rows_b[s].at[pl.ds(j * 128, 128)], semg_b[s]).start()
            for j in range(3):
                pltpu.make_async_copy(
                    nf_hbm.at[idx_b[s].at[pl.ds(j * 128, 128)]],
                    rows_b[s].at[pl.ds(j * 128, 128)], semg_b[s]).wait()
            pltpu.sync_copy(rows_b[s],
                            out_hbm.at[pl.ds(base_w + g * _G_GRP, _G_GRP)])

        fire_idx(0, 0)
        fire_idx(1, 1)

        @pl.loop(0, _G_NG - 1, step=2)
        def _(g):
            do_group(g, 0)
            fire_idx(g + 2, 0)
            do_group(g + 1, 1)

            @pl.when(g + 3 < _G_NG)
            def _():
                fire_idx(g + 3, 1)

        do_group(_G_NG - 1, 0)

        @pl.when(wid < 2)
        def _():
            tb = _NW * _G_SPAN + wid * 128
            pltpu.sync_copy(idx_hbm.at[pl.ds(tb, 128)], idx1.at[pl.ds(0, 128)])
            pltpu.sync_copy(nf_hbm.at[idx1.at[pl.ds(0, 128)]],
                            rows1.at[pl.ds(0, 128)])
            pltpu.sync_copy(rows1.at[pl.ds(0, 128)], out_hbm.at[pl.ds(tb, 128)])

    return k(node_feats, senders)


_BE = 1280


def _act(x):
    return jax.nn.silu(x) / _SILU_NORM


def _dgt(a, b):
    return lax.dot_general(a, b, (((0,), (0,)), ((), ())),
                           preferred_element_type=jnp.float32,
                           precision=lax.Precision.DEFAULT)


def _k2_body(ms_ref, rad_ref, vec_ref, w1_ref, w2_ref, w3_ref, w4_ref,
             out_ref):
    x = rad_ref[...]
    h = _act(_dgt(w1_ref[...], x))
    h = _act(_dgt(w2_ref[...], h))
    h = _act(_dgt(w3_ref[...], h)) * (1.0 / _AVG)

    v = -vec_ref[...]
    n2 = v[0:1, :] * v[0:1, :] + v[1:2, :] * v[1:2, :] + v[2:3, :] * v[2:3, :]
    inv = _SH_C / jnp.maximum(jnp.sqrt(n2), 1e-12)
    n = v * inv

    w4 = w4_ref[...]
    w4s = w4[:, 0:_D]
    w4v = w4[:, _D:]
    ms = ms_ref[...]
    out_ref[:, 0:_D] = ms * _dgt(h, w4s)
    out_ref[:, _D:2 * _D] = ms * _dgt(h * n[1:2, :], w4v)
    out_ref[:, 2 * _D:3 * _D] = ms * _dgt(h * n[2:3, :], w4v)
    out_ref[:, 3 * _D:4 * _D] = ms * _dgt(h * n[0:1, :], w4v)


def _k2_messages(msg_s, radial_t, vectors_t, W1, W2, W3, W4):
    grid = (_E // _BE,)
    return pl.pallas_call(
        _k2_body,
        grid=grid,
        in_specs=[
            pl.BlockSpec((_BE, _D), lambda i: (i, 0)),
            pl.BlockSpec((8, _BE), lambda i: (0, i)),
            pl.BlockSpec((3, _BE), lambda i: (0, i)),
            pl.BlockSpec((8, 64), lambda i: (0, 0)),
            pl.BlockSpec((64, 64), lambda i: (0, 0)),
            pl.BlockSpec((64, 64), lambda i: (0, 0)),
            pl.BlockSpec((64, 256), lambda i: (0, 0)),
        ],
        out_specs=pl.BlockSpec((_BE, 4 * _D), lambda i: (i, 0)),
        out_shape=jax.ShapeDtypeStruct((_E, 4 * _D), jnp.float32),
    )(msg_s, radial_t, vectors_t, W1, W2, W3, W4)


_SC_NS = 16
_C_W = 128
_NCHUNK = (4 * _D) // _C_W
_S_GRP = 128
_S_SPAN = 78 * 128
_S_NG = 78
_R_STRIPE = 624
_R_TAIL = _N - _SC_NS * _R_STRIPE
_Z_BLK = 48


def _k3_scatter(messages, receivers):
    mesh = plsc.VectorSubcoreMesh(core_axis_name="c", subcore_axis_name="s")

    @functools.partial(
        pl.kernel,
        out_type=jax.ShapeDtypeStruct((_N, 4 * _D), jnp.float32),
        mesh=mesh,
        scratch_types=[
            pltpu.VMEM((128,), jnp.int32),
            pltpu.VMEM((128,), jnp.int32),
            pltpu.VMEM((_S_GRP, _C_W), jnp.float32),
            pltpu.VMEM((_S_GRP, _C_W), jnp.float32),
            pltpu.VMEM((_Z_BLK, _C_W), jnp.float32),
            pltpu.VMEM_SHARED((_N, _C_W), jnp.float32),
            pltpu.SemaphoreType.DMA,
            pltpu.SemaphoreType.DMA,
            pltpu.SemaphoreType.DMA,
            pltpu.SemaphoreType.DMA,
        ],
    )
    def k(msg_hbm, rcv_hbm, out_hbm,
          ia0, ib0, dat0, dat1, zero_v, acc_sh,
          semi0, semi1, semd0, semd1):
        idx_b = (ia0, ib0)
        dat_b = (dat0, dat1)
        semi_b = (semi0, semi1)
        semd_b = (semd0, semd1)
        cid = lax.axis_index("c")
        sid = lax.axis_index("s")

        @pl.loop(0, _Z_BLK)
        def _(r):
            @pl.loop(0, _C_W, step=16)
            def _(cc):
                zero_v[r, pl.ds(cc, 16)] = jnp.zeros((16,), jnp.float32)

        row0 = sid * _R_STRIPE
        for qq in range(_NCHUNK // 2):
            q = cid * (_NCHUNK // 2) + qq
            col = q * _C_W

            def fire(g, s, col=col):
                b = sid * _S_SPAN + g * _S_GRP
                pltpu.make_async_copy(
                    rcv_hbm.at[pl.ds(b, _S_GRP)], idx_b[s], semi_b[s]).start()
                pltpu.make_async_copy(
                    msg_hbm.at[pl.ds(b, _S_GRP), pl.ds(col, _C_W)],
                    dat_b[s], semd_b[s]).start()

            def do_group(g, s, col=col):
                b = sid * _S_SPAN + g * _S_GRP
                pltpu.make_async_copy(
                    rcv_hbm.at[pl.ds(b, _S_GRP)], idx_b[s], semi_b[s]).wait()
                pltpu.make_async_copy(
                    msg_hbm.at[pl.ds(b, _S_GRP), pl.ds(col, _C_W)],
                    dat_b[s], semd_b[s]).wait()
                pltpu.sync_copy(dat_b[s], acc_sh.at[idx_b[s]], add=True)

            @pl.loop(0, _R_STRIPE // _Z_BLK)
            def _(zz):
                pltpu.sync_copy(zero_v, acc_sh.at[pl.ds(row0 + zz * _Z_BLK, _Z_BLK)])

            @pl.when(sid == 0)
            def _():
                pltpu.sync_copy(zero_v.at[pl.ds(0, _R_TAIL)],
                                acc_sh.at[pl.ds(_SC_NS * _R_STRIPE, _R_TAIL)])

            plsc.subcore_barrier()

            fire(0, 0)
            fire(1, 1)

            @pl.loop(0, _S_NG, step=2)
            def _(g):
                do_group(g, 0)

                @pl.when(g + 2 < _S_NG)
                def _():
                    fire(g + 2, 0)

                do_group(g + 1, 1)

                @pl.when(g + 3 < _S_NG)
                def _():
                    fire(g + 3, 1)

            @pl.when(sid < 2)
            def _():
                tb = _SC_NS * _S_SPAN + sid * 128
                pltpu.sync_copy(rcv_hbm.at[pl.ds(tb, 128)], ia0)
                pltpu.sync_copy(msg_hbm.at[pl.ds(tb, 128), pl.ds(col, _C_W)], dat0)
                pltpu.sync_copy(dat0, acc_sh.at[ia0], add=True)

            plsc.subcore_barrier()

            pltpu.sync_copy(
                acc_sh.at[pl.ds(row0, _R_STRIPE)],
                out_hbm.at[pl.ds(row0, _R_STRIPE), pl.ds(col, _C_W)],
            )

            @pl.when(sid == 0)
            def _():
                pltpu.sync_copy(
                    acc_sh.at[pl.ds(_SC_NS * _R_STRIPE, _R_TAIL)],
                    out_hbm.at[pl.ds(_SC_NS * _R_STRIPE, _R_TAIL), pl.ds(col, _C_W)],
                )

    return k(messages, receivers)


def _perm_matrix():
    p = np.zeros((3 * _D, 3 * _D), np.float32)
    for j in range(3):
        for d in range(_D):
            p[_D * j + d, 3 * d + j] = 1.0
    return p


_P = _perm_matrix()
_BR = 1000


def _k4_body(x_ref, p_ref, out_ref):
    out_ref[:, 0:_D] = x_ref[:, 0:_D]
    out_ref[:, _D:] = jnp.dot(x_ref[:, _D:], p_ref[...],
                              preferred_element_type=jnp.float32)


def _k4_permute(out_jm):
    grid = (_N // _BR,)
    return pl.pallas_call(
        _k4_body,
        grid=grid,
        in_specs=[
            pl.BlockSpec((_BR, 4 * _D), lambda i: (i, 0)),
            pl.BlockSpec((3 * _D, 3 * _D), lambda i: (0, 0)),
        ],
        out_specs=pl.BlockSpec((_BR, 4 * _D), lambda i: (i, 0)),
        out_shape=jax.ShapeDtypeStruct((_N, 4 * _D), jnp.float32),
    )(out_jm, jnp.asarray(_P))


def kernel(vectors, node_feats, radial_embedding, senders, receivers,
           W1, W2, W3, W4):
    assert node_feats.shape == (_N, _D) and senders.shape == (_E,)
    senders = senders.astype(jnp.int32)
    receivers = receivers.astype(jnp.int32)
    msg_s = _k1_gather(node_feats, senders)
    messages = _k2_messages(msg_s, radial_embedding.T, vectors.T,
                            W1, W2, W3, W4)
    out_jm = _k3_scatter(messages, receivers)
    return _k4_permute(out_jm)


# speedup vs baseline: 1.1041x; 1.1041x over previous
import functools

import numpy as np
import jax
import jax.numpy as jnp
from jax import lax
from jax.experimental import pallas as pl
from jax.experimental.pallas import tpu as pltpu
from jax.experimental.pallas import tpu_sc as plsc

_AVG = 16.0
_SILU_NORM = 0.5595081467
_SH_C = float(np.sqrt(3.0 / (4.0 * np.pi)))

_N = 10000
_E = 160000
_D = 128

_NW = 32
_SC_NS = 16
_BLK = 128

_E_A = 81920


def _pipe2(nblk, fire, do):
    fire(0, 0)
    fire(1, 1)

    @pl.loop(0, nblk // 2)
    def _(p):
        g = p * 2
        do(g, 0)

        @pl.when(g + 2 < nblk)
        def _():
            fire(g + 2, 0)

        do(g + 1, 1)

        @pl.when(g + 3 < nblk)
        def _():
            fire(g + 3, 1)

    if nblk % 2:
        do(nblk - 1, 0)


def _make_k1(e0, nblk_w, ntail):
    span = nblk_w * _BLK
    e_count = (nblk_w * _NW + ntail) * _BLK
    mesh = plsc.VectorSubcoreMesh(core_axis_name="c", subcore_axis_name="s")

    @functools.partial(
        pl.kernel,
        out_type=jax.ShapeDtypeStruct((e_count, _D), jnp.float32),
        mesh=mesh,
        scratch_types=[
            pltpu.VMEM((_BLK,), jnp.int32),
            pltpu.VMEM((_BLK,), jnp.int32),
            pltpu.VMEM((_BLK, _D), jnp.float32),
            pltpu.VMEM((_BLK, _D), jnp.float32),
            pltpu.SemaphoreType.DMA,
            pltpu.SemaphoreType.DMA,
        ],
    )
    def k(nf_hbm, idx_hbm, out_hbm, idx0, idx1, rows0, rows1, semi0, semi1):
        idx_b = (idx0, idx1)
        rows_b = (rows0, rows1)
        semi_b = (semi0, semi1)
        wid = lax.axis_index("s") * 2 + lax.axis_index("c")
        base_w = wid * span

        def fire(g, s):
            b = pl.multiple_of(base_w + g * _BLK, _BLK)
            pltpu.make_async_copy(idx_hbm.at[pl.ds(e0 + b, _BLK)],
                                  idx_b[s], semi_b[s]).start()

        def do(g, s):
            b = pl.multiple_of(base_w + g * _BLK, _BLK)
            pltpu.make_async_copy(idx_hbm.at[pl.ds(e0 + b, _BLK)],
                                  idx_b[s], semi_b[s]).wait()
            pltpu.sync_copy(nf_hbm.at[idx_b[s]], rows_b[s])
            pltpu.sync_copy(rows_b[s], out_hbm.at[pl.ds(b, _BLK)])

        _pipe2(nblk_w, fire, do)

        if ntail:
            @pl.when(wid < ntail)
            def _():
                tb = pl.multiple_of(_NW * span + wid * _BLK, _BLK)
                pltpu.sync_copy(idx_hbm.at[pl.ds(e0 + tb, _BLK)], idx0)
                pltpu.sync_copy(nf_hbm.at[idx0], rows0)
                pltpu.sync_copy(rows0, out_hbm.at[pl.ds(tb, _BLK)])

    return k


_BE = 1280


def _act(x):
    return jax.nn.silu(x) / _SILU_NORM


def _dgt(a, b):
    return lax.dot_general(a, b, (((0,), (0,)), ((), ())),
                           preferred_element_type=jnp.float32,
                           precision=lax.Precision.DEFAULT)


def _k2_body(ms_ref, rad_ref, vec_ref, w1_ref, w2_ref, w3_ref, w4_ref,
             out_ref):
    x = rad_ref[...]
    h = _act(_dgt(w1_ref[...], x))
    h = _act(_dgt(w2_ref[...], h))
    h = _act(_dgt(w3_ref[...], h)) * (1.0 / _AVG)

    v = -vec_ref[...]
    n2 = v[0:1, :] * v[0:1, :] + v[1:2, :] * v[1:2, :] + v[2:3, :] * v[2:3, :]
    inv = _SH_C / jnp.maximum(jnp.sqrt(n2), 1e-12)
    n = v * inv

    w4 = w4_ref[...]
    w4s = w4[:, 0:_D]
    w4v = w4[:, _D:]
    ms = ms_ref[...]
    out_ref[:, 0:_D] = ms * _dgt(h, w4s)
    out_ref[:, _D:2 * _D] = ms * _dgt(h * n[1:2, :], w4v)
    out_ref[:, 2 * _D:3 * _D] = ms * _dgt(h * n[2:3, :], w4v)
    out_ref[:, 3 * _D:4 * _D] = ms * _dgt(h * n[0:1, :], w4v)


def _k2_messages(msg_s, radial_t, vectors_t, W1, W2, W3, W4, e0):
    e_count = msg_s.shape[0]
    go = e0 // _BE
    grid = (e_count // _BE,)
    return pl.pallas_call(
        _k2_body,
        grid=grid,
        in_specs=[
            pl.BlockSpec((_BE, _D), lambda i: (i, 0)),
            pl.BlockSpec((8, _BE), lambda i: (0, i + go)),
            pl.BlockSpec((3, _BE), lambda i: (0, i + go)),
            pl.BlockSpec((8, 64), lambda i: (0, 0)),
            pl.BlockSpec((64, 64), lambda i: (0, 0)),
            pl.BlockSpec((64, 64), lambda i: (0, 0)),
            pl.BlockSpec((64, 256), lambda i: (0, 0)),
        ],
        out_specs=pl.BlockSpec((_BE, 4 * _D), lambda i: (i, 0)),
        out_shape=jax.ShapeDtypeStruct((e_count, 4 * _D), jnp.float32),
    )(msg_s, radial_t, vectors_t, W1, W2, W3, W4)


_C_W = 128
_NCHUNK = (4 * _D) // _C_W
_R_STRIPE = 624
_R_TAIL = _N - _SC_NS * _R_STRIPE
_Z_BLK = 48


def _make_k3(e0, nblk_s, ntail):
    span = nblk_s * _BLK
    mesh = plsc.VectorSubcoreMesh(core_axis_name="c", subcore_axis_name="s")

    @functools.partial(
        pl.kernel,
        out_type=jax.ShapeDtypeStruct((_N, 4 * _D), jnp.float32),
        mesh=mesh,
        scratch_types=[
            pltpu.VMEM((_BLK,), jnp.int32),
            pltpu.VMEM((_BLK,), jnp.int32),
            pltpu.VMEM((_BLK, _C_W), jnp.float32),
            pltpu.VMEM((_BLK, _C_W), jnp.float32),
            pltpu.VMEM((_Z_BLK, _C_W), jnp.float32),
            pltpu.VMEM_SHARED((_N, _C_W), jnp.float32),
            pltpu.SemaphoreType.DMA,
            pltpu.SemaphoreType.DMA,
            pltpu.SemaphoreType.DMA,
            pltpu.SemaphoreType.DMA,
        ],
    )
    def k(msg_hbm, rcv_hbm, out_hbm,
          ia0, ib0, dat0, dat1, zero_v, acc_sh,
          semi0, semi1, semd0, semd1):
        idx_b = (ia0, ib0)
        dat_b = (dat0, dat1)
        semi_b = (semi0, semi1)
        semd_b = (semd0, semd1)
        cid = lax.axis_index("c")
        sid = lax.axis_index("s")

        @pl.loop(0, _Z_BLK)
        def _(r):
            @pl.loop(0, _C_W, step=16)
            def _(cc):
                zero_v[r, pl.ds(cc, 16)] = jnp.zeros((16,), jnp.float32)

        row0 = pl.multiple_of(sid * _R_STRIPE, 16)
        for qq in range(_NCHUNK // 2):
            q = cid * (_NCHUNK // 2) + qq
            col = q * _C_W

            def fire(g, s, col=col):
                b = pl.multiple_of(sid * span + g * _BLK, _BLK)
                pltpu.make_async_copy(
                    rcv_hbm.at[pl.ds(e0 + b, _BLK)], idx_b[s], semi_b[s]).start()
                pltpu.make_async_copy(
                    msg_hbm.at[pl.ds(b, _BLK), pl.ds(col, _C_W)],
                    dat_b[s], semd_b[s]).start()

            def do_group(g, s, col=col):
                b = pl.multiple_of(sid * span + g * _BLK, _BLK)
                pltpu.make_async_copy(
                    rcv_hbm.at[pl.ds(e0 + b, _BLK)], idx_b[s], semi_b[s]).wait()
                pltpu.make_async_copy(
                    msg_hbm.at[pl.ds(b, _BLK), pl.ds(col, _C_W)],
                    dat_b[s], semd_b[s]).wait()
                pltpu.sync_copy(dat_b[s], acc_sh.at[idx_b[s]], add=True)

            @pl.loop(0, _R_STRIPE // _Z_BLK)
            def _(zz):
                zr = pl.multiple_of(row0 + zz * _Z_BLK, 16)
                pltpu.sync_copy(zero_v, acc_sh.at[pl.ds(zr, _Z_BLK)])

            @pl.when(sid == 0)
            def _():
                pltpu.sync_copy(zero_v.at[pl.ds(0, _R_TAIL)],
                                acc_sh.at[pl.ds(_SC_NS * _R_STRIPE, _R_TAIL)])

            plsc.subcore_barrier()

            _pipe2(nblk_s, fire, do_group)

            if ntail:
                @pl.when(sid < ntail)
                def _():
                    tb = pl.multiple_of(_SC_NS * span + sid * _BLK, _BLK)
                    pltpu.sync_copy(rcv_hbm.at[pl.ds(e0 + tb, _BLK)], ia0)
                    pltpu.sync_copy(
                        msg_hbm.at[pl.ds(tb, _BLK), pl.ds(col, _C_W)], dat0)
                    pltpu.sync_copy(dat0, acc_sh.at[ia0], add=True)

            plsc.subcore_barrier()

            pltpu.sync_copy(
                acc_sh.at[pl.ds(row0, _R_STRIPE)],
                out_hbm.at[pl.ds(row0, _R_STRIPE), pl.ds(col, _C_W)],
            )

            @pl.when(sid == 0)
            def _():
                pltpu.sync_copy(
                    acc_sh.at[pl.ds(_SC_NS * _R_STRIPE, _R_TAIL)],
                    out_hbm.at[pl.ds(_SC_NS * _R_STRIPE, _R_TAIL), pl.ds(col, _C_W)],
                )

    return k


def _perm_matrix():
    p = np.zeros((3 * _D, 3 * _D), np.float32)
    for j in range(3):
        for d in range(_D):
            p[_D * j + d, 3 * d + j] = 1.0
    return p


_P = _perm_matrix()
_BR = 1000


def _k4_body(a_ref, b_ref, p_ref, out_ref):
    x = a_ref[...] + b_ref[...]
    out_ref[:, 0:_D] = x[:, 0:_D]
    out_ref[:, _D:] = jnp.dot(x[:, _D:], p_ref[...],
                              preferred_element_type=jnp.float32)


def _k4_permute(out_a, out_b):
    grid = (_N // _BR,)
    return pl.pallas_call(
        _k4_body,
        grid=grid,
        in_specs=[
            pl.BlockSpec((_BR, 4 * _D), lambda i: (i, 0)),
            pl.BlockSpec((_BR, 4 * _D), lambda i: (i, 0)),
            pl.BlockSpec((3 * _D, 3 * _D), lambda i: (0, 0)),
        ],
        out_specs=pl.BlockSpec((_BR, 4 * _D), lambda i: (i, 0)),
        out_shape=jax.ShapeDtypeStruct((_N, 4 * _D), jnp.float32),
    )(out_a, out_b, jnp.asarray(_P))


def kernel(vectors, node_feats, radial_embedding, senders, receivers,
           W1, W2, W3, W4):
    assert node_feats.shape == (_N, _D) and senders.shape == (_E,)
    senders = senders.astype(jnp.int32)
    receivers = receivers.astype(jnp.int32)
    rad_t = radial_embedding.T
    vec_t = vectors.T

    k1a = _make_k1(0, 20, 0)
    k1b = _make_k1(_E_A, 19, 2)
    k3a = _make_k3(0, 40, 0)
    k3b = _make_k3(_E_A, 38, 2)

    ms_a = k1a(node_feats, senders)
    ms_b = k1b(node_feats, senders)
    mg_a = _k2_messages(ms_a, rad_t, vec_t, W1, W2, W3, W4, 0)
    mg_b = _k2_messages(ms_b, rad_t, vec_t, W1, W2, W3, W4, _E_A)
    out_a = k3a(mg_a, receivers)
    out_b = k3b(mg_b, receivers)
    return _k4_permute(out_a, out_b)
